# baseline (speedup 1.0000x reference)
import functools

import jax
import jax.numpy as jnp
from jax import lax
from jax.experimental import pallas as pl
from jax.experimental.pallas import tpu as pltpu
from jax.experimental.pallas import tpu_sc as plsc

_NC = 2
_NS = 16
_NW = _NC * _NS
_L16 = 16


@functools.lru_cache(maxsize=None)
def _make_emb(b, l, v, d):
    lp1 = l + 1
    bw = b // _NW
    eb_n = d // 8
    mesh = plsc.VectorSubcoreMesh(core_axis_name="c", subcore_axis_name="s")

    @functools.partial(
        pl.kernel,
        mesh=mesh,
        out_type=jax.ShapeDtypeStruct((lp1, eb_n, _NW, 8 * 128), jnp.float32),
        scratch_types=[
            pltpu.VMEM((bw, l), jnp.int32),
            pltpu.VMEM((2, bw), jnp.int32),
            pltpu.VMEM((2, bw, d), jnp.float32),
            pltpu.VMEM((2 * eb_n * 8 * 128,), jnp.float32),
            pltpu.SemaphoreType.DMA,
            pltpu.SemaphoreType.DMA,
            pltpu.SemaphoreType.DMA,
            pltpu.SemaphoreType.DMA,
        ],
        compiler_params=pltpu.CompilerParams(
            use_tc_tiling_on_sc=False, needs_layout_passes=False),
    )
    def emb(in_hbm, table_hbm, out_hbm, ids_v, idxc_v, rows_v, outt_v,
            g0, g1, o0, o1):
        wid = lax.axis_index("s") * _NC + lax.axis_index("c")
        sem_g = [g0, g1]
        sem_o = [o0, o1]

        pltpu.sync_copy(in_hbm.at[pl.ds(wid * bw, bw)], ids_v)

        iota = lax.iota(jnp.int32, _L16)
        pe = [((iota + _L16 * k) >> 3) * 1024 + ((iota + _L16 * k) & 7) * 128
              for k in range(d // _L16)]
        cls_vec = jnp.full((_L16,), v - 1, jnp.int32)

        def build_idxc(t, s):
            tm1 = jnp.full((_L16,), t - 1, jnp.int32)
            for k in range(bw // _L16):
                col = plsc.load_gather(ids_v, [iota + _L16 * k, tm1])
                idxc_v[s, pl.ds(_L16 * k, _L16)] = col

        def gather_copy(s):
            return pltpu.make_async_copy(
                table_hbm.at[idxc_v.at[s]], rows_v.at[s], sem_g[s])

        def out_tile_copy(t, eb, s):
            return pltpu.make_async_copy(
                outt_v.at[pl.ds((s * eb_n + eb) * 1024, 1024)],
                out_hbm.at[t, eb, wid], sem_o[s])

        def transpose_rows(s):
            for bs in range(bw):
                for k in range(d // _L16):
                    vals = rows_v[s, bs, pl.ds(_L16 * k, _L16)]
                    plsc.store_scatter(outt_v, [pe[k] + (s * 4096 + bs)],
                                       vals)

        def step(t, s):
            @pl.when(t >= 2)
            def _():
                for eb in range(eb_n):
                    out_tile_copy(t - 2, eb, s).wait()
            gather_copy(s).wait()
            @pl.when(t + 1 <= lp1 - 1)
            def _():
                build_idxc(t + 1, 1 - s)
                gather_copy(1 - s).start()
            transpose_rows(s)
            for eb in range(eb_n):
                out_tile_copy(t, eb, s).start()

        for k in range(bw // _L16):
            idxc_v[0, pl.ds(_L16 * k, _L16)] = cls_vec
        gather_copy(0).start()

        step(jnp.int32(0), 0)

        def body(tt, carry):
            step(1 + 2 * tt, 1)
            step(2 + 2 * tt, 0)
            return carry

        lax.fori_loop(0, (lp1 - 1) // 2, body, 0)

        for eb in range(eb_n):
            out_tile_copy(lp1 - 2, eb, 1).wait()
        for eb in range(eb_n):
            out_tile_copy(lp1 - 1, eb, 0).wait()

    return emb


def kernel(inputs, table):
    b, l = inputs.shape
    v, d = table.shape
    o5 = _make_emb(b, l, v, d)(inputs.astype(jnp.int32), table)
    return (o5.reshape(l + 1, d // 8, _NW, 8, 128)
            .transpose(2, 4, 0, 1, 3).reshape(b, l + 1, d))


# speedup vs baseline: 1.1009x; 1.1009x over previous
import functools

import jax
import jax.numpy as jnp
from jax import lax
from jax.experimental import pallas as pl
from jax.experimental.pallas import tpu as pltpu
from jax.experimental.pallas import tpu_sc as plsc

_NC = 2
_NS = 16
_NW = _NC * _NS
_BLK = 8


@functools.lru_cache(maxsize=None)
def _make_emb(b, l, v, d):
    per_w = b // _NW
    nblk = per_w // _BLK
    lp1 = l + 1
    mesh = plsc.VectorSubcoreMesh(core_axis_name="c", subcore_axis_name="s")

    @functools.partial(
        pl.kernel,
        mesh=mesh,
        out_type=jax.ShapeDtypeStruct((b, lp1, d), jnp.float32),
        scratch_types=[
            pltpu.VMEM((2, _BLK, l), jnp.int32),
            pltpu.VMEM((2, _BLK, lp1, d), jnp.float32),
            pltpu.VMEM((1, d), jnp.float32),
            pltpu.SemaphoreType.DMA,
            pltpu.SemaphoreType.DMA,
            pltpu.SemaphoreType.DMA,
            pltpu.SemaphoreType.DMA,
            pltpu.SemaphoreType.DMA,
        ],
        compiler_params=pltpu.CompilerParams(use_tc_tiling_on_sc=False),
    )
    def emb(in_hbm, table_hbm, cls_hbm, out_hbm, idx_v, rows_v, crow_v,
            sem_c, g0, g1, o0, o1):
        wid = lax.axis_index("s") * _NC + lax.axis_index("c")
        base = wid * per_w
        sem_g = [g0, g1]
        sem_o = [o0, o1]

        pltpu.async_copy(cls_hbm, crow_v, sem_c).wait()
        c0 = crow_v[0, pl.ds(0, 16)]
        c1 = crow_v[0, pl.ds(16, 16)]

        def gath(j, s):
            return pltpu.make_async_copy(
                table_hbm.at[idx_v.at[s, j]],
                rows_v.at[s, j, pl.ds(1, l)],
                sem_g[s])

        def out_copy(t, s):
            return pltpu.make_async_copy(
                rows_v.at[s],
                out_hbm.at[pl.ds(base + t * _BLK, _BLK)],
                sem_o[s])

        def body(tt, carry):
            for s in range(2):
                t = 2 * tt + s
                @pl.when(tt >= 1)
                def _():
                    out_copy(t - 2, s).wait()
                pltpu.sync_copy(in_hbm.at[pl.ds(base + t * _BLK, _BLK)],
                                idx_v.at[s])
                for j in range(_BLK):
                    rows_v[s, j, 0, pl.ds(0, 16)] = c0
                    rows_v[s, j, 0, pl.ds(16, 16)] = c1
                for j in range(_BLK):
                    gath(j, s).start()
                if s == 0:
                    @pl.when(tt >= 1)
                    def _():
                        for j in range(_BLK):
                            gath(j, 1).wait()
                        out_copy(t - 1, 1).start()
                else:
                    for j in range(_BLK):
                        gath(j, 0).wait()
                    out_copy(t - 1, 0).start()
            return carry

        lax.fori_loop(0, nblk // 2, body, 0)

        for j in range(_BLK):
            gath(j, 1).wait()
        out_copy(nblk - 1, 1).start()
        out_copy(nblk - 2, 0).wait()
        out_copy(nblk - 1, 1).wait()

    return emb


def kernel(inputs, table):
    b, l = inputs.shape
    v, d = table.shape
    return _make_emb(b, l, v, d)(
        inputs.astype(jnp.int32), table[:v - 1], table[v - 1:])
